# Initial kernel scaffold; baseline (speedup 1.0000x reference)
import jax
import jax.numpy as jnp
from jax.experimental import pallas as pl


def kernel(positions, table):
    raise NotImplementedError("write your pallas kernel here")


# speedup vs baseline: 1.6105x; 1.6105x over previous
import functools

import jax
import jax.numpy as jnp
from jax import lax
from jax.experimental import pallas as pl
from jax.experimental.pallas import tpu as pltpu
from jax.experimental.pallas import tpu_sc as plsc

D_MODEL = 2048
NC = 2
NS = 16
NW = NC * NS
B = 4 * 8192
B_PER_W = B // NW
CHUNK = 16
NCHUNK = B_PER_W // CHUNK

_mesh = plsc.VectorSubcoreMesh(
    core_axis_name="c", subcore_axis_name="s", num_cores=NC, num_subcores=NS
)


@functools.partial(
    pl.kernel,
    mesh=_mesh,
    out_type=jax.ShapeDtypeStruct((B, D_MODEL), jnp.float32),
    scratch_types=[
        pltpu.VMEM((B_PER_W,), jnp.int32),
        pltpu.VMEM((CHUNK, D_MODEL), jnp.float32),
        pltpu.VMEM((CHUNK, D_MODEL), jnp.float32),
        pltpu.SemaphoreType.DMA,
        pltpu.SemaphoreType.DMA,
    ],
)
def _gather_rows(table_hbm, idx_hbm, out_hbm, idx_v, rows0, rows1, sem0, sem1):
    rows = (rows0, rows1)
    sems = (sem0, sem1)
    wid = lax.axis_index("s") * NC + lax.axis_index("c")
    base = wid * B_PER_W
    pltpu.sync_copy(idx_hbm.at[pl.ds(base, B_PER_W)], idx_v)

    def gather_start(c, b):
        pltpu.async_copy(
            table_hbm.at[idx_v.at[pl.ds(c * CHUNK, CHUNK)]], rows[b], sems[b]
        )

    def gather_wait(b):
        pltpu.make_async_copy(table_hbm.at[pl.ds(0, CHUNK)], rows[b], sems[b]).wait()

    def write_out(c, b):
        pltpu.sync_copy(rows[b], out_hbm.at[pl.ds(base + c * CHUNK, CHUNK)])

    gather_start(0, 0)
    gather_start(1, 1)

    @pl.loop(0, NCHUNK - 2, step=2)
    def _(c):
        for b in range(2):
            gather_wait(b)
            write_out(c + b, b)
            gather_start(c + b + 2, b)

    for b in range(2):
        gather_wait(b)
        write_out(NCHUNK - 2 + b, b)


def kernel(positions, table):
    idx = positions.reshape(-1).astype(jnp.int32)
    out = _gather_rows(table, idx)
    return out.reshape(*positions.shape, D_MODEL)
